# baseline (device time: 36222 ns/iter reference)
import jax
import jax.numpy as jnp
from jax import lax
from jax.experimental import pallas as pl
from jax.experimental.pallas import tpu as pltpu

B, S, H, Dh, Dr = 2, 256, 16, 64, 32
D = 1024
DC = 64
G = 4
HG = H // G
GC = HG * Dh
GR = HG * Dr
SCALE = (Dh + Dr) ** -0.5


def kernel(x, Wdkv, Wuk, Wuv, Wq, Wqr, Wkr, Wo):

    def body(x_h, wdkv_h, wuk_h, wuv_h, wq_h, wqr_h, wkr_h, wo_h, out_h,
             x_v, wdkv_v, wuk_v, wuv_v, wq_v, wqr_v, wkr_v, wo_v, out_v,
             c_loc, c_rem, kvw_loc, kvw_rem, o_parts,
             load_sems, store_sems, x_send_sems, x_recv_sems,
             o_send_sems, o_recv_sems):
        my_x = lax.axis_index("x")
        my_y = lax.axis_index("y")
        my_z = lax.axis_index("z")
        my_g = 2 * my_y + my_z
        peer_x = (1 - my_x, my_y, my_z)
        o_peers = [
            (my_x, 1 - my_y, my_z),
            (my_x, my_y, 1 - my_z),
            (my_x, 1 - my_y, 1 - my_z),
        ]
        slot_g = [
            2 * (1 - my_y) + my_z,
            2 * my_y + (1 - my_z),
            2 * (1 - my_y) + (1 - my_z),
            my_g,
        ]

        loads = []
        for i, (h, v) in enumerate([
                (x_h, x_v), (wdkv_h, wdkv_v), (wuk_h, wuk_v), (wuv_h, wuv_v),
                (wq_h, wq_v), (wqr_h, wqr_v), (wkr_h, wkr_v), (wo_h, wo_v)]):
            cp = pltpu.make_async_copy(h, v, load_sems.at[i])
            cp.start()
            loads.append(cp)

        barrier = pltpu.get_barrier_semaphore()
        for nbr in [peer_x] + o_peers:
            pl.semaphore_signal(barrier, inc=1, device_id=nbr,
                                device_id_type=pl.DeviceIdType.MESH)
        pl.semaphore_wait(barrier, 4)

        for i in range(4):
            loads[i].wait()

        wdkv = wdkv_v[...].astype(jnp.bfloat16)
        kvw_loc[0] = wuk_v[:, pl.ds(my_g * GC, GC)].astype(jnp.bfloat16)
        kvw_loc[1] = wuv_v[:, pl.ds(my_g * GC, GC)].astype(jnp.bfloat16)
        for b in range(B):
            c_loc[b] = jnp.dot(x_v[b].astype(jnp.bfloat16), wdkv,
                               preferred_element_type=jnp.float32
                               ).astype(jnp.bfloat16)

        rdma_c = pltpu.make_async_remote_copy(
            src_ref=c_loc, dst_ref=c_rem,
            send_sem=x_send_sems.at[0], recv_sem=x_recv_sems.at[0],
            device_id=peer_x, device_id_type=pl.DeviceIdType.MESH)
        rdma_w = pltpu.make_async_remote_copy(
            src_ref=kvw_loc, dst_ref=kvw_rem,
            send_sem=x_send_sems.at[1], recv_sem=x_recv_sems.at[1],
            device_id=peer_x, device_id_type=pl.DeviceIdType.MESH)
        rdma_c.start()
        rdma_w.start()

        loads[4].wait()
        loads[5].wait()
        loads[6].wait()
        wq_g = wq_v[:, pl.ds(my_g * GC, GC)].astype(jnp.bfloat16)
        wqr_g = wqr_v[:, pl.ds(my_g * GR, GR)].astype(jnp.bfloat16)
        wkr = wkr_v[...].astype(jnp.bfloat16)
        Q = []
        Qr = []
        Kr = []
        Kp = []
        Vp = []
        for b in range(B):
            xb = x_v[b].astype(jnp.bfloat16)
            Q.append(jnp.dot(xb, wq_g,
                             preferred_element_type=jnp.float32
                             ).astype(jnp.bfloat16))
            Qr.append(jnp.dot(xb, wqr_g,
                              preferred_element_type=jnp.float32
                              ).astype(jnp.bfloat16))
            Kr.append(jnp.dot(xb, wkr,
                              preferred_element_type=jnp.float32
                              ).astype(jnp.bfloat16))
            Kp.append(jnp.dot(c_loc[b], kvw_loc[0],
                              preferred_element_type=jnp.float32))
            Vp.append(jnp.dot(c_loc[b], kvw_loc[1],
                              preferred_element_type=jnp.float32))

        rdma_c.wait()
        rdma_w.wait()

        for b in range(B):
            Kb = (Kp[b] + jnp.dot(c_rem[b], kvw_rem[0],
                                  preferred_element_type=jnp.float32)
                  ).astype(jnp.bfloat16)
            Vb = (Vp[b] + jnp.dot(c_rem[b], kvw_rem[1],
                                  preferred_element_type=jnp.float32)
                  ).astype(jnp.bfloat16)
            for h in range(HG):
                q = Q[b][:, h * Dh:(h + 1) * Dh]
                k = Kb[:, h * Dh:(h + 1) * Dh]
                v = Vb[:, h * Dh:(h + 1) * Dh]
                qr = Qr[b][:, h * Dr:(h + 1) * Dr]
                s = (lax.dot_general(q, k, (((1,), (1,)), ((), ())),
                                     preferred_element_type=jnp.float32)
                     + lax.dot_general(qr, Kr[b], (((1,), (1,)), ((), ())),
                                       preferred_element_type=jnp.float32)
                     ) * SCALE
                m = jnp.max(s, axis=-1, keepdims=True)
                p = jnp.exp(s - m)
                p = p / jnp.sum(p, axis=-1, keepdims=True)
                o_parts[3, b, :, h * Dh:(h + 1) * Dh] = jnp.dot(
                    p.astype(jnp.bfloat16), v,
                    preferred_element_type=jnp.float32).astype(jnp.bfloat16)

        o_rdmas = []
        for j, nbr in enumerate(o_peers):
            r = pltpu.make_async_remote_copy(
                src_ref=o_parts.at[3], dst_ref=o_parts.at[j],
                send_sem=o_send_sems.at[j], recv_sem=o_recv_sems.at[j],
                device_id=nbr, device_id_type=pl.DeviceIdType.MESH)
            r.start()
            o_rdmas.append(r)

        loads[7].wait()
        for b in range(B):
            out_v[b] = jnp.dot(
                o_parts[3, b],
                wo_v[pl.ds(my_g * GC, GC), :].astype(jnp.bfloat16),
                preferred_element_type=jnp.float32)

        for r in o_rdmas:
            r.wait()

        stores = []
        for b in range(B):
            for j in range(3):
                wo_j = wo_v[pl.ds(slot_g[j] * GC, GC), :].astype(jnp.bfloat16)
                out_v[b] += jnp.dot(o_parts[j, b], wo_j,
                                    preferred_element_type=jnp.float32)
            st = pltpu.make_async_copy(out_v.at[b], out_h.at[b],
                                       store_sems.at[b])
            st.start()
            stores.append(st)
        for st in stores:
            st.wait()

    return pl.pallas_call(
        body,
        out_shape=jax.ShapeDtypeStruct((B, S, D), jnp.float32),
        in_specs=[pl.BlockSpec(memory_space=pl.ANY)] * 8,
        out_specs=pl.BlockSpec(memory_space=pl.ANY),
        scratch_shapes=[
            pltpu.VMEM((B, S, D), jnp.float32),
            pltpu.VMEM((D, DC), jnp.float32),
            pltpu.VMEM((DC, D), jnp.float32),
            pltpu.VMEM((DC, D), jnp.float32),
            pltpu.VMEM((D, D), jnp.float32),
            pltpu.VMEM((D, H * Dr), jnp.float32),
            pltpu.VMEM((D, Dr), jnp.float32),
            pltpu.VMEM((D, D), jnp.float32),
            pltpu.VMEM((B, S, D), jnp.float32),
            pltpu.VMEM((B, S, DC), jnp.bfloat16),
            pltpu.VMEM((B, S, DC), jnp.bfloat16),
            pltpu.VMEM((2, DC, GC), jnp.bfloat16),
            pltpu.VMEM((2, DC, GC), jnp.bfloat16),
            pltpu.VMEM((4, B, S, GC), jnp.bfloat16),
            pltpu.SemaphoreType.DMA((8,)),
            pltpu.SemaphoreType.DMA((B,)),
            pltpu.SemaphoreType.DMA((2,)),
            pltpu.SemaphoreType.DMA((2,)),
            pltpu.SemaphoreType.DMA((3,)),
            pltpu.SemaphoreType.DMA((3,)),
        ],
        compiler_params=pltpu.CompilerParams(collective_id=0),
    )(x, Wdkv, Wuk, Wuv, Wq, Wqr, Wkr, Wo)


# device time: 28770 ns/iter; 1.2590x vs baseline; 1.2590x over previous
import jax
import jax.numpy as jnp
from jax import lax
from jax.experimental import pallas as pl
from jax.experimental.pallas import tpu as pltpu

B, S, H, Dh, Dr = 2, 256, 16, 64, 32
D = 1024
DC = 64
G = 4
HG = H // G
GC = HG * Dh
GR = HG * Dr
SCALE = (Dh + Dr) ** -0.5


def kernel(x, Wdkv, Wuk, Wuv, Wq, Wqr, Wkr, Wo):

    def body(x_h, wdkv_h, wuk_h, wuv_h, wq_h, wqr_h, wkr_h, wo_h, out_h,
             x_v, wdkv_v, wuk_v, wuv_v, wq_v, wqr_v, wkr_v, wo_v, out_v,
             c_loc, c_rem, kvw_loc, kvw_rem, o_parts,
             load_sems, store_sems, x_send_sems, x_recv_sems,
             o_send_sems, o_recv_sems):
        my_x = lax.axis_index("x")
        my_y = lax.axis_index("y")
        my_z = lax.axis_index("z")
        my_g = 2 * my_y + my_z
        peer_x = (1 - my_x, my_y, my_z)
        o_peers = [
            (my_x, 1 - my_y, my_z),
            (my_x, my_y, 1 - my_z),
            (my_x, 1 - my_y, 1 - my_z),
        ]
        slot_g = [
            2 * (1 - my_y) + my_z,
            2 * my_y + (1 - my_z),
            2 * (1 - my_y) + (1 - my_z),
            my_g,
        ]

        loads = []
        for i, (h, v) in enumerate([
                (x_h, x_v), (wdkv_h, wdkv_v), (wuk_h, wuk_v), (wuv_h, wuv_v),
                (wq_h, wq_v), (wqr_h, wqr_v), (wkr_h, wkr_v), (wo_h, wo_v)]):
            cp = pltpu.make_async_copy(h, v, load_sems.at[i])
            cp.start()
            loads.append(cp)

        barrier = pltpu.get_barrier_semaphore()
        for nbr in [peer_x] + o_peers:
            pl.semaphore_signal(barrier, inc=1, device_id=nbr,
                                device_id_type=pl.DeviceIdType.MESH)
        pl.semaphore_wait(barrier, 4)

        for i in range(4):
            loads[i].wait()

        wdkv = wdkv_v[...].astype(jnp.bfloat16)
        kvw_loc[0] = wuk_v[:, pl.ds(my_g * GC, GC)].astype(jnp.bfloat16)
        kvw_loc[1] = wuv_v[:, pl.ds(my_g * GC, GC)].astype(jnp.bfloat16)
        for b in range(B):
            c_loc[b] = jnp.dot(x_v[b].astype(jnp.bfloat16), wdkv,
                               preferred_element_type=jnp.float32
                               ).astype(jnp.bfloat16)

        rdma_c = pltpu.make_async_remote_copy(
            src_ref=c_loc, dst_ref=c_rem,
            send_sem=x_send_sems.at[0], recv_sem=x_recv_sems.at[0],
            device_id=peer_x, device_id_type=pl.DeviceIdType.MESH)
        rdma_w = pltpu.make_async_remote_copy(
            src_ref=kvw_loc, dst_ref=kvw_rem,
            send_sem=x_send_sems.at[1], recv_sem=x_recv_sems.at[1],
            device_id=peer_x, device_id_type=pl.DeviceIdType.MESH)
        rdma_c.start()
        rdma_w.start()

        loads[4].wait()
        loads[5].wait()
        loads[6].wait()
        wq_g = wq_v[:, pl.ds(my_g * GC, GC)].astype(jnp.bfloat16)
        wqr_g = wqr_v[:, pl.ds(my_g * GR, GR)].astype(jnp.bfloat16)
        wkr = wkr_v[...].astype(jnp.bfloat16)
        Q = []
        Qr = []
        Kr = []
        Kp = []
        Vp = []
        for b in range(B):
            xb = x_v[b].astype(jnp.bfloat16)
            Q.append(jnp.dot(xb, wq_g,
                             preferred_element_type=jnp.float32
                             ).astype(jnp.bfloat16))
            Qr.append(jnp.dot(xb, wqr_g,
                              preferred_element_type=jnp.float32
                              ).astype(jnp.bfloat16))
            Kr.append(jnp.dot(xb, wkr,
                              preferred_element_type=jnp.float32
                              ).astype(jnp.bfloat16))
            Kp.append(jnp.dot(c_loc[b], kvw_loc[0],
                              preferred_element_type=jnp.float32))
            Vp.append(jnp.dot(c_loc[b], kvw_loc[1],
                              preferred_element_type=jnp.float32))

        rdma_c.wait()
        rdma_w.wait()

        for b in range(B):
            Kb = (Kp[b] + jnp.dot(c_rem[b], kvw_rem[0],
                                  preferred_element_type=jnp.float32)
                  ).astype(jnp.bfloat16)
            Vb = (Vp[b] + jnp.dot(c_rem[b], kvw_rem[1],
                                  preferred_element_type=jnp.float32)
                  ).astype(jnp.bfloat16)
            for h in range(HG):
                q = Q[b][:, h * Dh:(h + 1) * Dh]
                k = Kb[:, h * Dh:(h + 1) * Dh]
                v = Vb[:, h * Dh:(h + 1) * Dh]
                qr = Qr[b][:, h * Dr:(h + 1) * Dr]
                s = (lax.dot_general(q, k, (((1,), (1,)), ((), ())),
                                     preferred_element_type=jnp.float32)
                     + lax.dot_general(qr, Kr[b], (((1,), (1,)), ((), ())),
                                       preferred_element_type=jnp.float32)
                     ) * SCALE
                m = jnp.max(s, axis=-1, keepdims=True)
                p = jnp.exp(s - m)
                p = p / jnp.sum(p, axis=-1, keepdims=True)
                o_parts[3, b, :, h * Dh:(h + 1) * Dh] = jnp.dot(
                    p.astype(jnp.bfloat16), v,
                    preferred_element_type=jnp.float32).astype(jnp.bfloat16)

        o_rdmas = []
        for j, nbr in enumerate(o_peers):
            r = pltpu.make_async_remote_copy(
                src_ref=o_parts.at[3], dst_ref=o_parts.at[j],
                send_sem=o_send_sems.at[j], recv_sem=o_recv_sems.at[j],
                device_id=nbr, device_id_type=pl.DeviceIdType.MESH)
            r.start()
            o_rdmas.append(r)

        loads[7].wait()
        for b in range(B):
            out_v[b] = jnp.dot(
                o_parts[3, b],
                wo_v[pl.ds(my_g * GC, GC), :].astype(jnp.bfloat16),
                preferred_element_type=jnp.float32)

        for r in o_rdmas:
            r.wait()

        stores = []
        for b in range(B):
            for j in range(3):
                wo_j = wo_v[pl.ds(slot_g[j] * GC, GC), :].astype(jnp.bfloat16)
                out_v[b] += jnp.dot(o_parts[j, b], wo_j,
                                    preferred_element_type=jnp.float32)
            st = pltpu.make_async_copy(out_v.at[b], out_h.at[b],
                                       store_sems.at[b])
            st.start()
            stores.append(st)
        for st in stores:
            st.wait()

    args = [pltpu.with_memory_space_constraint(a, pltpu.MemorySpace.HBM)
            for a in (x, Wdkv, Wuk, Wuv, Wq, Wqr, Wkr, Wo)]

    return pl.pallas_call(
        body,
        out_shape=jax.ShapeDtypeStruct((B, S, D), jnp.float32),
        in_specs=[pl.BlockSpec(memory_space=pl.ANY)] * 8,
        out_specs=pl.BlockSpec(memory_space=pl.ANY),
        scratch_shapes=[
            pltpu.VMEM((B, S, D), jnp.float32),
            pltpu.VMEM((D, DC), jnp.float32),
            pltpu.VMEM((DC, D), jnp.float32),
            pltpu.VMEM((DC, D), jnp.float32),
            pltpu.VMEM((D, D), jnp.float32),
            pltpu.VMEM((D, H * Dr), jnp.float32),
            pltpu.VMEM((D, Dr), jnp.float32),
            pltpu.VMEM((D, D), jnp.float32),
            pltpu.VMEM((B, S, D), jnp.float32),
            pltpu.VMEM((B, S, DC), jnp.bfloat16),
            pltpu.VMEM((B, S, DC), jnp.bfloat16),
            pltpu.VMEM((2, DC, GC), jnp.bfloat16),
            pltpu.VMEM((2, DC, GC), jnp.bfloat16),
            pltpu.VMEM((4, B, S, GC), jnp.bfloat16),
            pltpu.SemaphoreType.DMA((8,)),
            pltpu.SemaphoreType.DMA((B,)),
            pltpu.SemaphoreType.DMA((2,)),
            pltpu.SemaphoreType.DMA((2,)),
            pltpu.SemaphoreType.DMA((3,)),
            pltpu.SemaphoreType.DMA((3,)),
        ],
        compiler_params=pltpu.CompilerParams(collective_id=0),
    )(*args)


# device time: 25631 ns/iter; 1.4132x vs baseline; 1.1225x over previous
import jax
import jax.numpy as jnp
from jax import lax
from jax.experimental import pallas as pl
from jax.experimental.pallas import tpu as pltpu

B, S, H, Dh, Dr = 2, 256, 16, 64, 32
D = 1024
DC = 64
G = 4
HG = H // G
GC = HG * Dh
GR = HG * Dr
SCALE = (Dh + Dr) ** -0.5


def kernel(x, Wdkv, Wuk, Wuv, Wq, Wqr, Wkr, Wo):

    def body(x_h, wdkv_h, wuk_h, wuv_h, wq_h, wqr_h, wkr_h, wo_h, out_h,
             x_v, wdkv_v, wuk_v, wuv_v, wq_v, wqr_v, wkr_v, wo_v, out_v,
             c_loc, c_rem, kvw_loc, kvw_rem, o_parts,
             load_sems, store_sems, x_send_sems, x_recv_sems,
             o_send_sems, o_recv_sems):
        my_x = lax.axis_index("x")
        my_y = lax.axis_index("y")
        my_z = lax.axis_index("z")
        my_g = 2 * my_y + my_z
        peer_x = (1 - my_x, my_y, my_z)
        o_peers = [
            (my_x, 1 - my_y, my_z),
            (my_x, my_y, 1 - my_z),
            (my_x, 1 - my_y, 1 - my_z),
        ]
        slot_g = [
            2 * (1 - my_y) + my_z,
            2 * my_y + (1 - my_z),
            2 * (1 - my_y) + (1 - my_z),
            my_g,
        ]

        loads = []
        for i, (h, v) in enumerate([
                (x_h, x_v), (wdkv_h, wdkv_v), (wuk_h, wuk_v), (wuv_h, wuv_v),
                (wq_h, wq_v), (wqr_h, wqr_v), (wkr_h, wkr_v), (wo_h, wo_v)]):
            cp = pltpu.make_async_copy(h, v, load_sems.at[i])
            cp.start()
            loads.append(cp)

        barrier = pltpu.get_barrier_semaphore()
        for nbr in [peer_x] + o_peers:
            pl.semaphore_signal(barrier, inc=1, device_id=nbr,
                                device_id_type=pl.DeviceIdType.MESH)
        pl.semaphore_wait(barrier, 4)

        for i in range(4):
            loads[i].wait()

        wdkv = wdkv_v[...].astype(jnp.bfloat16)
        kvw_loc[0] = wuk_v[:, pl.ds(my_g * GC, GC)].astype(jnp.bfloat16)
        kvw_loc[1] = wuv_v[:, pl.ds(my_g * GC, GC)].astype(jnp.bfloat16)
        for b in range(B):
            c_loc[b] = jnp.dot(x_v[b].astype(jnp.bfloat16), wdkv,
                               preferred_element_type=jnp.float32
                               ).astype(jnp.bfloat16)

        rdma_c = pltpu.make_async_remote_copy(
            src_ref=c_loc, dst_ref=c_rem,
            send_sem=x_send_sems.at[0], recv_sem=x_recv_sems.at[0],
            device_id=peer_x, device_id_type=pl.DeviceIdType.MESH)
        rdma_w = pltpu.make_async_remote_copy(
            src_ref=kvw_loc, dst_ref=kvw_rem,
            send_sem=x_send_sems.at[1], recv_sem=x_recv_sems.at[1],
            device_id=peer_x, device_id_type=pl.DeviceIdType.MESH)
        rdma_c.start()
        rdma_w.start()

        loads[4].wait()
        loads[5].wait()
        loads[6].wait()
        wq_g = wq_v[:, pl.ds(my_g * GC, GC)].astype(jnp.bfloat16)
        wqr_g = wqr_v[:, pl.ds(my_g * GR, GR)].astype(jnp.bfloat16)
        wkr = wkr_v[...].astype(jnp.bfloat16)
        Q = []
        Qr = []
        Kr = []
        Kp = []
        Vp = []
        for b in range(B):
            xb = x_v[b].astype(jnp.bfloat16)
            Q.append(jnp.dot(xb, wq_g,
                             preferred_element_type=jnp.float32
                             ).astype(jnp.bfloat16))
            Qr.append(jnp.dot(xb, wqr_g,
                              preferred_element_type=jnp.float32
                              ).astype(jnp.bfloat16))
            Kr.append(jnp.dot(xb, wkr,
                              preferred_element_type=jnp.float32
                              ).astype(jnp.bfloat16))
            Kp.append(jnp.dot(c_loc[b], kvw_loc[0],
                              preferred_element_type=jnp.float32))
            Vp.append(jnp.dot(c_loc[b], kvw_loc[1],
                              preferred_element_type=jnp.float32))

        rdma_c.wait()
        rdma_w.wait()

        o_rdmas = []
        for b in range(B):
            Kb = (Kp[b] + jnp.dot(c_rem[b], kvw_rem[0],
                                  preferred_element_type=jnp.float32)
                  ).astype(jnp.bfloat16)
            Vb = (Vp[b] + jnp.dot(c_rem[b], kvw_rem[1],
                                  preferred_element_type=jnp.float32)
                  ).astype(jnp.bfloat16)
            for h in range(HG):
                q = Q[b][:, h * Dh:(h + 1) * Dh]
                k = Kb[:, h * Dh:(h + 1) * Dh]
                v = Vb[:, h * Dh:(h + 1) * Dh]
                qr = Qr[b][:, h * Dr:(h + 1) * Dr]
                s = (lax.dot_general(q, k, (((1,), (1,)), ((), ())),
                                     preferred_element_type=jnp.float32)
                     + lax.dot_general(qr, Kr[b], (((1,), (1,)), ((), ())),
                                       preferred_element_type=jnp.float32)
                     ) * SCALE
                m = jnp.max(s, axis=-1, keepdims=True)
                p = jnp.exp(s - m)
                p = p / jnp.sum(p, axis=-1, keepdims=True)
                o_parts[3, b, :, h * Dh:(h + 1) * Dh] = jnp.dot(
                    p.astype(jnp.bfloat16), v,
                    preferred_element_type=jnp.float32).astype(jnp.bfloat16)
            for j, nbr in enumerate(o_peers):
                r = pltpu.make_async_remote_copy(
                    src_ref=o_parts.at[3, b], dst_ref=o_parts.at[j, b],
                    send_sem=o_send_sems.at[j, b],
                    recv_sem=o_recv_sems.at[j, b],
                    device_id=nbr, device_id_type=pl.DeviceIdType.MESH)
                r.start()
                o_rdmas.append((j, b, r))

        loads[7].wait()
        for b in range(B):
            out_v[b] = jnp.dot(
                o_parts[3, b],
                wo_v[pl.ds(my_g * GC, GC), :].astype(jnp.bfloat16),
                preferred_element_type=jnp.float32)

        for j in range(3):
            wo_j = wo_v[pl.ds(slot_g[j] * GC, GC), :].astype(jnp.bfloat16)
            for jj, b, r in o_rdmas:
                if jj == j:
                    r.wait()
                    out_v[b] += jnp.dot(o_parts[j, b], wo_j,
                                        preferred_element_type=jnp.float32)

        stores = []
        for b in range(B):
            st = pltpu.make_async_copy(out_v.at[b], out_h.at[b],
                                       store_sems.at[b])
            st.start()
            stores.append(st)
        for st in stores:
            st.wait()

    args = [pltpu.with_memory_space_constraint(a, pltpu.MemorySpace.HBM)
            for a in (x, Wdkv, Wuk, Wuv, Wq, Wqr, Wkr, Wo)]

    return pl.pallas_call(
        body,
        out_shape=jax.ShapeDtypeStruct((B, S, D), jnp.float32),
        in_specs=[pl.BlockSpec(memory_space=pl.ANY)] * 8,
        out_specs=pl.BlockSpec(memory_space=pltpu.MemorySpace.HBM),
        scratch_shapes=[
            pltpu.VMEM((B, S, D), jnp.float32),
            pltpu.VMEM((D, DC), jnp.float32),
            pltpu.VMEM((DC, D), jnp.float32),
            pltpu.VMEM((DC, D), jnp.float32),
            pltpu.VMEM((D, D), jnp.float32),
            pltpu.VMEM((D, H * Dr), jnp.float32),
            pltpu.VMEM((D, Dr), jnp.float32),
            pltpu.VMEM((D, D), jnp.float32),
            pltpu.VMEM((B, S, D), jnp.float32),
            pltpu.VMEM((B, S, DC), jnp.bfloat16),
            pltpu.VMEM((B, S, DC), jnp.bfloat16),
            pltpu.VMEM((2, DC, GC), jnp.bfloat16),
            pltpu.VMEM((2, DC, GC), jnp.bfloat16),
            pltpu.VMEM((4, B, S, GC), jnp.bfloat16),
            pltpu.SemaphoreType.DMA((8,)),
            pltpu.SemaphoreType.DMA((B,)),
            pltpu.SemaphoreType.DMA((2,)),
            pltpu.SemaphoreType.DMA((2,)),
            pltpu.SemaphoreType.DMA((3, B)),
            pltpu.SemaphoreType.DMA((3, B)),
        ],
        compiler_params=pltpu.CompilerParams(collective_id=0),
    )(*args)
